# baseline (speedup 1.0000x reference)
import functools

import jax
import jax.numpy as jnp
from jax import lax
from jax.experimental import pallas as pl
from jax.experimental.pallas import tpu as pltpu
from jax.experimental.pallas import tpu_sc as plsc

L = 16
NC = 2
NS = 16
NW = NC * NS
IDX_CHUNK = 128
TC_BLOCK = 2048


def _sc_gather(ids_a, ids_b, table2):
    batch = ids_a.shape[0]
    rw2 = table2.shape[1]
    b_per_w = batch // NW
    n_chunks = b_per_w // IDX_CHUNK

    mesh = plsc.VectorSubcoreMesh(core_axis_name="c", subcore_axis_name="s")

    @functools.partial(
        pl.kernel,
        out_type=(
            jax.ShapeDtypeStruct((batch, rw2), jnp.float32),
            jax.ShapeDtypeStruct((batch, rw2), jnp.float32),
        ),
        mesh=mesh,
        scratch_types=[
            pltpu.VMEM((b_per_w,), jnp.int32),
            pltpu.VMEM((b_per_w,), jnp.int32),
            pltpu.VMEM((2, IDX_CHUNK, rw2), jnp.float32),
            pltpu.VMEM((2, IDX_CHUNK, rw2), jnp.float32),
            pltpu.SemaphoreType.DMA,
            pltpu.SemaphoreType.DMA,
        ],
    )
    def sc_kernel(ids_a_hbm, ids_b_hbm, table_hbm, ra_out, rb_out,
                  idxa_v, idxb_v, ga_v, gb_v, sem, out_sem):
        wid = lax.axis_index("s") * NC + lax.axis_index("c")
        base = wid * b_per_w
        pltpu.sync_copy(ids_a_hbm.at[pl.ds(base, b_per_w)], idxa_v)
        pltpu.sync_copy(ids_b_hbm.at[pl.ds(base, b_per_w)], idxb_v)
        for k in range(b_per_w // L):
            sl = pl.ds(k * L, L)
            idxa_v[sl] = lax.shift_right_logical(idxa_v[sl], 1)
            idxb_v[sl] = lax.shift_right_logical(idxb_v[sl], 1)

        def fire(j):
            buf = j % 2
            isl = pl.ds(j * IDX_CHUNK, IDX_CHUNK)
            return (
                pltpu.async_copy(table_hbm.at[idxa_v.at[isl]], ga_v.at[buf], sem),
                pltpu.async_copy(table_hbm.at[idxb_v.at[isl]], gb_v.at[buf], sem),
            )

        pend_out = {}
        cps = fire(0)
        for j in range(n_chunks):
            nbuf = (j + 1) % 2
            nxt = None
            if j + 1 < n_chunks:
                for cp in pend_out.pop(nbuf, ()):
                    cp.wait()
                nxt = fire(j + 1)
            for cp in cps:
                cp.wait()
            buf = j % 2
            dst = pl.ds(base + j * IDX_CHUNK, IDX_CHUNK)
            pend_out[buf] = (
                pltpu.async_copy(ga_v.at[buf], ra_out.at[dst], out_sem),
                pltpu.async_copy(gb_v.at[buf], rb_out.at[dst], out_sem),
            )
            cps = nxt
        for lst in pend_out.values():
            for cp in lst:
                cp.wait()

    return sc_kernel(ids_a, ids_b, table2)


def _tc_probs(par_a, par_b, rows_a, rows_b, dim):
    batch, rw2 = rows_a.shape

    def body(pa_ref, pb_ref, ra_ref, rb_ref, p_ref, a_ref, b_ref):
        ra_lo = ra_ref[:, : rw2 // 2]
        ra_hi = ra_ref[:, rw2 // 2:]
        rb_lo = rb_ref[:, : rw2 // 2]
        rb_hi = rb_ref[:, rw2 // 2:]
        sa = ra_lo + pa_ref[...] * (ra_hi - ra_lo)
        sb = rb_lo + pb_ref[...] * (rb_hi - rb_lo)
        za, ha = sa[:, :dim], sa[:, dim:]
        zb, hb = sb[:, :dim], sb[:, dim:]
        w = jnp.maximum(jnp.minimum(ha, hb) - jnp.maximum(za, zb), 0.0)
        s = jnp.maximum(ha - za, 0.0)
        q = w / s
        while q.shape[1] > 1:
            h = q.shape[1] // 2
            q = q[:, :h] * q[:, h:]
        p_ref[...] = q
        a_ref[...] = sa
        b_ref[...] = sb

    n_blk = batch // TC_BLOCK
    return pl.pallas_call(
        body,
        grid=(n_blk,),
        in_specs=[
            pl.BlockSpec((TC_BLOCK, 1), lambda i: (i, 0)),
            pl.BlockSpec((TC_BLOCK, 1), lambda i: (i, 0)),
            pl.BlockSpec((TC_BLOCK, rw2), lambda i: (i, 0)),
            pl.BlockSpec((TC_BLOCK, rw2), lambda i: (i, 0)),
        ],
        out_specs=[
            pl.BlockSpec((TC_BLOCK, 1), lambda i: (i, 0)),
            pl.BlockSpec((TC_BLOCK, rw2 // 2), lambda i: (i, 0)),
            pl.BlockSpec((TC_BLOCK, rw2 // 2), lambda i: (i, 0)),
        ],
        out_shape=(
            jax.ShapeDtypeStruct((batch, 1), jnp.float32),
            jax.ShapeDtypeStruct((batch, rw2 // 2), jnp.float32),
            jax.ShapeDtypeStruct((batch, rw2 // 2), jnp.float32),
        ),
    )(par_a, par_b, rows_a, rows_b)


def kernel(ids, boxes):
    num_models, num_boxes, two, dim = boxes.shape
    batch = ids.shape[1]
    table2 = boxes.reshape(num_boxes // 2, 4 * dim)
    ids_a, ids_b = ids[0], ids[1]
    rows_a, rows_b = _sc_gather(ids_a, ids_b, table2)
    par_a = (ids_a & 1).astype(jnp.float32).reshape(batch, 1)
    par_b = (ids_b & 1).astype(jnp.float32).reshape(batch, 1)
    p, a, b = _tc_probs(par_a, par_b, rows_a, rows_b, dim)
    return (p.reshape(num_models, batch),
            a.reshape(num_models, batch, two, dim),
            b.reshape(num_models, batch, two, dim))


# speedup vs baseline: 5.9099x; 5.9099x over previous
import functools

import jax
import jax.numpy as jnp
from jax import lax
from jax.experimental import pallas as pl
from jax.experimental.pallas import tpu as pltpu
from jax.experimental.pallas import tpu_sc as plsc

L = 16
NC = 2
NS = 16
NW = NC * NS
IDX_CHUNK = 128
TC_BLOCK = 2048


def _sc_gather(ids_a, ids_b, table2, h2):
    batch = ids_a.shape[0]
    rw2 = table2.shape[1]
    b_per_w = batch // NW
    n_chunks = b_per_w // IDX_CHUNK

    mesh = plsc.VectorSubcoreMesh(core_axis_name="c", subcore_axis_name="s")

    @functools.partial(
        pl.kernel,
        out_type=(
            jax.ShapeDtypeStruct((batch, rw2), jnp.float32),
            jax.ShapeDtypeStruct((batch, rw2), jnp.float32),
        ),
        mesh=mesh,
        scratch_types=[
            pltpu.VMEM((b_per_w,), jnp.int32),
            pltpu.VMEM((b_per_w,), jnp.int32),
            pltpu.VMEM((2, IDX_CHUNK, rw2), jnp.float32),
            pltpu.VMEM((2, IDX_CHUNK, rw2), jnp.float32),
            pltpu.SemaphoreType.DMA,
            pltpu.SemaphoreType.DMA,
        ],
    )
    def sc_kernel(ids_a_hbm, ids_b_hbm, table_hbm, ra_out, rb_out,
                  idxa_v, idxb_v, ga_v, gb_v, sem, out_sem):
        wid = lax.axis_index("s") * NC + lax.axis_index("c")
        base = wid * b_per_w
        pltpu.sync_copy(ids_a_hbm.at[pl.ds(base, b_per_w)], idxa_v)
        pltpu.sync_copy(ids_b_hbm.at[pl.ds(base, b_per_w)], idxb_v)
        for k in range(b_per_w // L):
            sl = pl.ds(k * L, L)
            va = idxa_v[sl]
            vb = idxb_v[sl]
            idxa_v[sl] = jnp.where(va >= h2, va - h2, va)
            idxb_v[sl] = jnp.where(vb >= h2, vb - h2, vb)

        def fire(j):
            buf = j % 2
            isl = pl.ds(j * IDX_CHUNK, IDX_CHUNK)
            return (
                pltpu.async_copy(table_hbm.at[idxa_v.at[isl]], ga_v.at[buf], sem),
                pltpu.async_copy(table_hbm.at[idxb_v.at[isl]], gb_v.at[buf], sem),
            )

        pend_out = {}
        cps = fire(0)
        for j in range(n_chunks):
            nbuf = (j + 1) % 2
            nxt = None
            if j + 1 < n_chunks:
                for cp in pend_out.pop(nbuf, ()):
                    cp.wait()
                nxt = fire(j + 1)
            for cp in cps:
                cp.wait()
            buf = j % 2
            dst = pl.ds(base + j * IDX_CHUNK, IDX_CHUNK)
            pend_out[buf] = (
                pltpu.async_copy(ga_v.at[buf], ra_out.at[dst], out_sem),
                pltpu.async_copy(gb_v.at[buf], rb_out.at[dst], out_sem),
            )
            cps = nxt
        for lst in pend_out.values():
            for cp in lst:
                cp.wait()

    return sc_kernel(ids_a, ids_b, table2)


def _tc_probs(par_a, par_b, rows_a, rows_b, dim):
    batch, rw2 = rows_a.shape

    def body(pa_ref, pb_ref, ra_ref, rb_ref, p_ref, a_ref, b_ref):
        ra_lo = ra_ref[:, : rw2 // 2]
        ra_hi = ra_ref[:, rw2 // 2:]
        rb_lo = rb_ref[:, : rw2 // 2]
        rb_hi = rb_ref[:, rw2 // 2:]
        sa = ra_lo + pa_ref[...] * (ra_hi - ra_lo)
        sb = rb_lo + pb_ref[...] * (rb_hi - rb_lo)
        za, ha = sa[:, :dim], sa[:, dim:]
        zb, hb = sb[:, :dim], sb[:, dim:]
        w = jnp.maximum(jnp.minimum(ha, hb) - jnp.maximum(za, zb), 0.0)
        s = jnp.maximum(ha - za, 0.0)
        q = w / s
        while q.shape[1] > 1:
            h = q.shape[1] // 2
            q = q[:, :h] * q[:, h:]
        p_ref[...] = q
        a_ref[...] = sa
        b_ref[...] = sb

    n_blk = batch // TC_BLOCK
    return pl.pallas_call(
        body,
        grid=(n_blk,),
        in_specs=[
            pl.BlockSpec((TC_BLOCK, 1), lambda i: (i, 0)),
            pl.BlockSpec((TC_BLOCK, 1), lambda i: (i, 0)),
            pl.BlockSpec((TC_BLOCK, rw2), lambda i: (i, 0)),
            pl.BlockSpec((TC_BLOCK, rw2), lambda i: (i, 0)),
        ],
        out_specs=[
            pl.BlockSpec((TC_BLOCK, 1), lambda i: (i, 0)),
            pl.BlockSpec((TC_BLOCK, rw2 // 2), lambda i: (i, 0)),
            pl.BlockSpec((TC_BLOCK, rw2 // 2), lambda i: (i, 0)),
        ],
        out_shape=(
            jax.ShapeDtypeStruct((batch, 1), jnp.float32),
            jax.ShapeDtypeStruct((batch, rw2 // 2), jnp.float32),
            jax.ShapeDtypeStruct((batch, rw2 // 2), jnp.float32),
        ),
    )(par_a, par_b, rows_a, rows_b)


def _tc_transpose(table_nat):
    rw, v = table_nat.shape
    cb = 2048
    n_blk = (v + 2 * cb - 1) // (2 * cb)
    h2 = n_blk * cb
    last_blk = (v + cb - 1) // cb - 1

    def body(lo_ref, hi_ref, out_ref):
        out_ref[:, :rw] = lo_ref[...].T
        out_ref[:, rw:] = hi_ref[...].T

    out = pl.pallas_call(
        body,
        grid=(n_blk,),
        in_specs=[
            pl.BlockSpec((rw, cb), lambda i: (0, i)),
            pl.BlockSpec((rw, cb), lambda i: (0, jnp.minimum(i + n_blk, last_blk))),
        ],
        out_specs=pl.BlockSpec((cb, 2 * rw), lambda i: (i, 0)),
        out_shape=jax.ShapeDtypeStruct((h2, 2 * rw), jnp.float32),
    )(table_nat, table_nat)
    return out, h2


def kernel(ids, boxes):
    num_models, num_boxes, two, dim = boxes.shape
    batch = ids.shape[1]
    table_nat = jnp.transpose(boxes, (0, 2, 3, 1)).reshape(2 * dim, num_boxes)
    table2, h2 = _tc_transpose(table_nat)
    ids_a, ids_b = ids[0], ids[1]
    rows_a, rows_b = _sc_gather(ids_a, ids_b, table2, h2)
    par_a = (ids_a >= h2).astype(jnp.float32).reshape(batch, 1)
    par_b = (ids_b >= h2).astype(jnp.float32).reshape(batch, 1)
    p, a, b = _tc_probs(par_a, par_b, rows_a, rows_b, dim)
    return (p.reshape(num_models, batch),
            a.reshape(num_models, batch, two, dim),
            b.reshape(num_models, batch, two, dim))


# speedup vs baseline: 7.7866x; 1.3176x over previous
import functools

import jax
import jax.numpy as jnp
from jax import lax
from jax.experimental import pallas as pl
from jax.experimental.pallas import tpu as pltpu
from jax.experimental.pallas import tpu_sc as plsc

L = 16
NC = 2
NS = 16
NW = NC * NS
IDX_CHUNK = 128
TC_BLOCK = 2048


def _sc_gather(ids_a, ids_b, table2, h2):
    batch = ids_a.shape[0]
    rw2 = table2.shape[1]
    b_per_w = batch // NW
    n_chunks = b_per_w // IDX_CHUNK

    mesh = plsc.VectorSubcoreMesh(core_axis_name="c", subcore_axis_name="s")

    @functools.partial(
        pl.kernel,
        out_type=(
            jax.ShapeDtypeStruct((batch, rw2), jnp.float32),
            jax.ShapeDtypeStruct((batch, rw2), jnp.float32),
        ),
        mesh=mesh,
        scratch_types=[
            pltpu.VMEM((b_per_w,), jnp.int32),
            pltpu.VMEM((b_per_w,), jnp.int32),
            pltpu.VMEM((2, IDX_CHUNK, rw2), jnp.float32),
            pltpu.VMEM((2, IDX_CHUNK, rw2), jnp.float32),
            pltpu.SemaphoreType.DMA,
            pltpu.SemaphoreType.DMA,
        ],
    )
    def sc_kernel(ids_a_hbm, ids_b_hbm, table_hbm, ra_out, rb_out,
                  idxa_v, idxb_v, ga_v, gb_v, sem, out_sem):
        wid = lax.axis_index("s") * NC + lax.axis_index("c")
        base = wid * b_per_w
        pltpu.sync_copy(ids_a_hbm.at[pl.ds(base, b_per_w)], idxa_v)
        pltpu.sync_copy(ids_b_hbm.at[pl.ds(base, b_per_w)], idxb_v)
        for k in range(b_per_w // L):
            sl = pl.ds(k * L, L)
            va = idxa_v[sl]
            vb = idxb_v[sl]
            idxa_v[sl] = jnp.where(va >= h2, va - h2, va)
            idxb_v[sl] = jnp.where(vb >= h2, vb - h2, vb)

        def fire(j):
            buf = j % 2
            isl = pl.ds(j * IDX_CHUNK, IDX_CHUNK)
            return (
                pltpu.async_copy(table_hbm.at[idxa_v.at[isl]], ga_v.at[buf], sem),
                pltpu.async_copy(table_hbm.at[idxb_v.at[isl]], gb_v.at[buf], sem),
            )

        pend_out = {}
        cps = fire(0)
        for j in range(n_chunks):
            nbuf = (j + 1) % 2
            nxt = None
            if j + 1 < n_chunks:
                for cp in pend_out.pop(nbuf, ()):
                    cp.wait()
                nxt = fire(j + 1)
            for cp in cps:
                cp.wait()
            buf = j % 2
            dst = pl.ds(base + j * IDX_CHUNK, IDX_CHUNK)
            pend_out[buf] = (
                pltpu.async_copy(ga_v.at[buf], ra_out.at[dst], out_sem),
                pltpu.async_copy(gb_v.at[buf], rb_out.at[dst], out_sem),
            )
            cps = nxt
        for lst in pend_out.values():
            for cp in lst:
                cp.wait()

    return sc_kernel(ids_a, ids_b, table2)


def _tc_probs(par_a, par_b, rows_a, rows_b, dim):
    batch, rw2 = rows_a.shape

    def body(pa_ref, pb_ref, ra_ref, rb_ref, p_ref, a_ref, b_ref):
        ra_lo = ra_ref[:, : rw2 // 2]
        ra_hi = ra_ref[:, rw2 // 2:]
        rb_lo = rb_ref[:, : rw2 // 2]
        rb_hi = rb_ref[:, rw2 // 2:]
        sa = ra_lo + pa_ref[...] * (ra_hi - ra_lo)
        sb = rb_lo + pb_ref[...] * (rb_hi - rb_lo)
        za, ha = sa[:, :dim], sa[:, dim:]
        zb, hb = sb[:, :dim], sb[:, dim:]
        w = jnp.maximum(jnp.minimum(ha, hb) - jnp.maximum(za, zb), 0.0)
        s = jnp.maximum(ha - za, 0.0)
        q = w / s
        while q.shape[1] > 1:
            h = q.shape[1] // 2
            q = q[:, :h] * q[:, h:]
        p_ref[...] = q
        a_ref[...] = sa
        b_ref[...] = sb

    n_blk = batch // TC_BLOCK
    return pl.pallas_call(
        body,
        grid=(n_blk,),
        in_specs=[
            pl.BlockSpec((TC_BLOCK, 1), lambda i: (i, 0)),
            pl.BlockSpec((TC_BLOCK, 1), lambda i: (i, 0)),
            pl.BlockSpec((TC_BLOCK, rw2), lambda i: (i, 0)),
            pl.BlockSpec((TC_BLOCK, rw2), lambda i: (i, 0)),
        ],
        out_specs=[
            pl.BlockSpec((TC_BLOCK, 1), lambda i: (i, 0)),
            pl.BlockSpec((TC_BLOCK, rw2 // 2), lambda i: (i, 0)),
            pl.BlockSpec((TC_BLOCK, rw2 // 2), lambda i: (i, 0)),
        ],
        out_shape=(
            jax.ShapeDtypeStruct((batch, 1), jnp.float32),
            jax.ShapeDtypeStruct((batch, rw2 // 2), jnp.float32),
            jax.ShapeDtypeStruct((batch, rw2 // 2), jnp.float32),
        ),
    )(par_a, par_b, rows_a, rows_b)


def _tc_transpose(table_nat):
    rw, v = table_nat.shape
    cb = 8192
    n_blk = (v + 2 * cb - 1) // (2 * cb)
    h2 = n_blk * cb
    last_blk = (v + cb - 1) // cb - 1

    def body(lo_ref, hi_ref, out_ref):
        out_ref[:, :rw] = lo_ref[...].T
        out_ref[:, rw:] = hi_ref[...].T

    out = pl.pallas_call(
        body,
        grid=(n_blk,),
        in_specs=[
            pl.BlockSpec((rw, cb), lambda i: (0, i)),
            pl.BlockSpec((rw, cb), lambda i: (0, jnp.minimum(i + n_blk, last_blk))),
        ],
        out_specs=pl.BlockSpec((cb, 2 * rw), lambda i: (i, 0)),
        out_shape=jax.ShapeDtypeStruct((h2, 2 * rw), jnp.float32),
    )(table_nat, table_nat)
    return out, h2


def kernel(ids, boxes):
    num_models, num_boxes, two, dim = boxes.shape
    batch = ids.shape[1]
    table_nat = jnp.transpose(boxes, (0, 2, 3, 1)).reshape(2 * dim, num_boxes)
    table2, h2 = _tc_transpose(table_nat)
    ids_a, ids_b = ids[0], ids[1]
    rows_a, rows_b = _sc_gather(ids_a, ids_b, table2, h2)
    par_a = (ids_a >= h2).astype(jnp.float32).reshape(batch, 1)
    par_b = (ids_b >= h2).astype(jnp.float32).reshape(batch, 1)
    p, a, b = _tc_probs(par_a, par_b, rows_a, rows_b, dim)
    return (p.reshape(num_models, batch),
            a.reshape(num_models, batch, two, dim),
            b.reshape(num_models, batch, two, dim))


# speedup vs baseline: 8.1130x; 1.0419x over previous
import functools

import jax
import jax.numpy as jnp
from jax import lax
from jax.experimental import pallas as pl
from jax.experimental.pallas import tpu as pltpu
from jax.experimental.pallas import tpu_sc as plsc

L = 16
NC = 2
NS = 16
NW = NC * NS
IDX_CHUNK = 128
TC_BLOCK = 2048


def _sc_gather(ids_a, ids_b, table2, h2):
    batch = ids_a.shape[0]
    rw2 = table2.shape[1]
    b_per_w = batch // NW
    n_chunks = b_per_w // IDX_CHUNK

    mesh = plsc.VectorSubcoreMesh(core_axis_name="c", subcore_axis_name="s")

    @functools.partial(
        pl.kernel,
        out_type=(
            jax.ShapeDtypeStruct((batch, rw2), jnp.float32),
            jax.ShapeDtypeStruct((batch, rw2), jnp.float32),
        ),
        mesh=mesh,
        scratch_types=[
            pltpu.VMEM((b_per_w,), jnp.int32),
            pltpu.VMEM((b_per_w,), jnp.int32),
            pltpu.VMEM((2, IDX_CHUNK, rw2), jnp.float32),
            pltpu.VMEM((2, IDX_CHUNK, rw2), jnp.float32),
            pltpu.SemaphoreType.DMA,
            pltpu.SemaphoreType.DMA,
        ],
    )
    def sc_kernel(ids_a_hbm, ids_b_hbm, table_hbm, ra_out, rb_out,
                  idxa_v, idxb_v, ga_v, gb_v, sem, out_sem):
        wid = lax.axis_index("s") * NC + lax.axis_index("c")
        base = wid * b_per_w
        pltpu.sync_copy(ids_a_hbm.at[pl.ds(base, b_per_w)], idxa_v)
        pltpu.sync_copy(ids_b_hbm.at[pl.ds(base, b_per_w)], idxb_v)
        for k in range(b_per_w // L):
            sl = pl.ds(k * L, L)
            va = idxa_v[sl]
            vb = idxb_v[sl]
            idxa_v[sl] = jnp.where(va >= h2, va - h2, va)
            idxb_v[sl] = jnp.where(vb >= h2, vb - h2, vb)

        def fire(j):
            buf = j % 2
            isl = pl.ds(j * IDX_CHUNK, IDX_CHUNK)
            return (
                pltpu.async_copy(table_hbm.at[idxa_v.at[isl]], ga_v.at[buf], sem),
                pltpu.async_copy(table_hbm.at[idxb_v.at[isl]], gb_v.at[buf], sem),
            )

        pend_out = {}
        cps = fire(0)
        for j in range(n_chunks):
            nbuf = (j + 1) % 2
            nxt = None
            if j + 1 < n_chunks:
                for cp in pend_out.pop(nbuf, ()):
                    cp.wait()
                nxt = fire(j + 1)
            for cp in cps:
                cp.wait()
            buf = j % 2
            dst = pl.ds(base + j * IDX_CHUNK, IDX_CHUNK)
            pend_out[buf] = (
                pltpu.async_copy(ga_v.at[buf], ra_out.at[dst], out_sem),
                pltpu.async_copy(gb_v.at[buf], rb_out.at[dst], out_sem),
            )
            cps = nxt
        for lst in pend_out.values():
            for cp in lst:
                cp.wait()

    return sc_kernel(ids_a, ids_b, table2)


def _tc_probs(par_a, par_b, rows_a, rows_b, dim):
    batch, rw2 = rows_a.shape

    def body(pa_ref, pb_ref, ra_ref, rb_ref, p_ref, a_ref, b_ref):
        ra_lo = ra_ref[:, : rw2 // 2]
        ra_hi = ra_ref[:, rw2 // 2:]
        rb_lo = rb_ref[:, : rw2 // 2]
        rb_hi = rb_ref[:, rw2 // 2:]
        sa = ra_lo + pa_ref[...] * (ra_hi - ra_lo)
        sb = rb_lo + pb_ref[...] * (rb_hi - rb_lo)
        za, ha = sa[:, :dim], sa[:, dim:]
        zb, hb = sb[:, :dim], sb[:, dim:]
        w = jnp.maximum(jnp.minimum(ha, hb) - jnp.maximum(za, zb), 0.0)
        s = jnp.maximum(ha - za, 0.0)
        q = w / s
        while q.shape[1] > 1:
            h = q.shape[1] // 2
            q = q[:, :h] * q[:, h:]
        p_ref[...] = q.T
        a_ref[...] = sa.T
        b_ref[...] = sb.T

    n_blk = batch // TC_BLOCK
    return pl.pallas_call(
        body,
        grid=(n_blk,),
        in_specs=[
            pl.BlockSpec((TC_BLOCK, 1), lambda i: (i, 0)),
            pl.BlockSpec((TC_BLOCK, 1), lambda i: (i, 0)),
            pl.BlockSpec((TC_BLOCK, rw2), lambda i: (i, 0)),
            pl.BlockSpec((TC_BLOCK, rw2), lambda i: (i, 0)),
        ],
        out_specs=[
            pl.BlockSpec((1, TC_BLOCK), lambda i: (0, i)),
            pl.BlockSpec((rw2 // 2, TC_BLOCK), lambda i: (0, i)),
            pl.BlockSpec((rw2 // 2, TC_BLOCK), lambda i: (0, i)),
        ],
        out_shape=(
            jax.ShapeDtypeStruct((1, batch), jnp.float32),
            jax.ShapeDtypeStruct((rw2 // 2, batch), jnp.float32),
            jax.ShapeDtypeStruct((rw2 // 2, batch), jnp.float32),
        ),
    )(par_a, par_b, rows_a, rows_b)


def _tc_transpose(table_nat):
    rw, v = table_nat.shape
    cb = 8192
    n_blk = (v + 2 * cb - 1) // (2 * cb)
    h2 = n_blk * cb
    last_blk = (v + cb - 1) // cb - 1

    def body(lo_ref, hi_ref, out_ref):
        out_ref[:, :rw] = lo_ref[...].T
        out_ref[:, rw:] = hi_ref[...].T

    out = pl.pallas_call(
        body,
        grid=(n_blk,),
        in_specs=[
            pl.BlockSpec((rw, cb), lambda i: (0, i)),
            pl.BlockSpec((rw, cb), lambda i: (0, jnp.minimum(i + n_blk, last_blk))),
        ],
        out_specs=pl.BlockSpec((cb, 2 * rw), lambda i: (i, 0)),
        out_shape=jax.ShapeDtypeStruct((h2, 2 * rw), jnp.float32),
    )(table_nat, table_nat)
    return out, h2


def kernel(ids, boxes):
    num_models, num_boxes, two, dim = boxes.shape
    batch = ids.shape[1]
    table_nat = jnp.transpose(boxes, (0, 2, 3, 1)).reshape(2 * dim, num_boxes)
    table2, h2 = _tc_transpose(table_nat)
    ids_a, ids_b = ids[0], ids[1]
    rows_a, rows_b = _sc_gather(ids_a, ids_b, table2, h2)
    par_a = (ids_a >= h2).astype(jnp.float32).reshape(batch, 1)
    par_b = (ids_b >= h2).astype(jnp.float32).reshape(batch, 1)
    p, a, b = _tc_probs(par_a, par_b, rows_a, rows_b, dim)
    a = jnp.transpose(a.reshape(two, dim, batch), (2, 0, 1))
    b = jnp.transpose(b.reshape(two, dim, batch), (2, 0, 1))
    return (p.reshape(num_models, batch),
            a.reshape(num_models, batch, two, dim),
            b.reshape(num_models, batch, two, dim))
